# baseline (speedup 1.0000x reference)
import functools

import jax
import jax.numpy as jnp
from jax import lax
from jax.experimental import pallas as pl
from jax.experimental.pallas import tpu as pltpu
from jax.experimental.pallas import tpu_sc as plsc

N = 10000
NP = 10112
NC, NS = 2, 16
NW = NC * NS
K = 128
NEG = -1e30


def _expand_a(a):
    H, C = a.shape
    M = jnp.zeros((H, C, H), jnp.float32).at[jnp.arange(H), :, jnp.arange(H)].set(a)
    return M.reshape(H * C, H)


def _head_expand(H, C):
    R = jnp.zeros((H, H, C), jnp.float32).at[jnp.arange(H), jnp.arange(H), :].set(1.0)
    return R.reshape(H, H * C)


def _prep_tables(xw, asrc, adst, H, HC, DS):
    row = lax.broadcasted_iota(jnp.int32, (NP, H), 0)
    asrc = jnp.where(row < N, asrc, NEG)
    gmax = jnp.max(asrc, axis=0, keepdims=True)
    t = gmax + adst
    bsub = jnp.where(t >= 0, t, 0.2 * t)
    def cat(parts):
        parts = [p for p in parts if p.shape[1] > 0]
        return parts[0] if len(parts) == 1 else jnp.concatenate(parts, axis=1)

    S = cat([asrc, jnp.zeros((NP, 16 - H), jnp.float32), xw,
             jnp.zeros((NP, DS - 16 - HC), jnp.float32)])
    D = cat([adst, bsub, jnp.zeros((NP, 16 - 2 * H), jnp.float32)])
    return S, D


def _tc_prep1(x_ref, w_ref, as_ref, ad_ref, p_ref, s_ref, d_ref, *, H, HC, DS):
    xw = jnp.dot(x_ref[...], w_ref[...], preferred_element_type=jnp.float32)
    asrc = jnp.dot(xw, as_ref[...], preferred_element_type=jnp.float32)
    adst = jnp.dot(xw, ad_ref[...], preferred_element_type=jnp.float32)
    S, D = _prep_tables(xw, asrc, adst, H, HC, DS)
    s_ref[...] = jnp.dot(S, p_ref[...],
                         preferred_element_type=jnp.float32).astype(jnp.bfloat16)
    d_ref[...] = D


def _tc_mid(acc_ref, r_ref, b_ref, w_ref, as_ref, ad_ref, p_ref, s_ref, d_ref,
            *, HCp, DENC, H, HC, DS):
    msum = acc_ref[0, :, 0:HCp] + acc_ref[1, :, 0:HCp]
    den = acc_ref[0, :, DENC:DENC + 8] + acc_ref[1, :, DENC:DENC + 8]
    den_e = jnp.dot(den, r_ref[...], preferred_element_type=jnp.float32)
    x = msum / (den_e + 1e-16) + b_ref[...]
    xw = jnp.dot(x, w_ref[...], preferred_element_type=jnp.float32)
    asrc = jnp.dot(xw, as_ref[...], preferred_element_type=jnp.float32)
    adst = jnp.dot(xw, ad_ref[...], preferred_element_type=jnp.float32)
    S, D = _prep_tables(xw, asrc, adst, H, HC, DS)
    s_ref[...] = jnp.dot(S, p_ref[...],
                         preferred_element_type=jnp.float32).astype(jnp.bfloat16)
    d_ref[...] = D


def _tc_final(acc_ref, b_ref, o_ref):
    msum = acc_ref[0, :, 0:40] + acc_ref[1, :, 0:40]
    den = acc_ref[0, :, 48:49] + acc_ref[1, :, 48:49]
    x = msum / (den + 1e-16) + b_ref[...]
    m = jnp.max(x, axis=1, keepdims=True)
    o_ref[...] = x - (m + jnp.log(jnp.sum(jnp.exp(x - m), axis=1, keepdims=True)))


def _make_sc_edge(E_pad, DS, DA, NXV, CW, HL):
    NG = DS // 32
    CPW = E_pad // (NW * K)
    RPT = NP // NS
    DENC = NXV * 16
    mesh = plsc.VectorSubcoreMesh(core_axis_name="c", subcore_axis_name="s")

    @functools.partial(
        pl.kernel,
        out_type=jax.ShapeDtypeStruct((NC, NP, DA), jnp.float32),
        mesh=mesh,
        compiler_params=pltpu.CompilerParams(use_tc_tiling_on_sc=False,
                                             needs_layout_passes=False),
        scratch_types=[
            pltpu.VMEM((CPW, 2, K), jnp.int32),
            pltpu.VMEM((2, K, DS), jnp.bfloat16),
            pltpu.VMEM((2, K, 16), jnp.float32),
            pltpu.VMEM((2, K, DA), jnp.float32),
            pltpu.VMEM_SHARED((NP, DA), jnp.float32),
            pltpu.SemaphoreType.DMA,
            pltpu.SemaphoreType.DMA,
            pltpu.SemaphoreType.DMA,
            pltpu.SemaphoreType.DMA,
            pltpu.SemaphoreType.DMA,
            pltpu.SemaphoreType.DMA,
        ],
    )
    def sc_edge(s_hbm, d_hbm, idx_hbm, z_hbm, out_hbm,
                sd, srow, drow, msg, acc, ss0, ss1, sd0, sd1, sw0, sw1):
        cid = lax.axis_index("c")
        sid = lax.axis_index("s")
        wid = cid * NS + sid
        lane = lax.iota(jnp.int32, 16)
        bsidx = HL + lane % HL
        bcidx = [(16 * k + lane) // CW for k in range(NXV)]
        headm = lane < HL
        semS = [ss0, ss1]
        semD = [sd0, sd1]
        semW = [sw0, sw1]
        gd = lax.GatherDimensionNumbers(offset_dims=(), collapsed_slice_dims=(0,),
                                        start_index_map=(0,))

        def dg(x, idx):
            return lax.gather(x, idx[:, None], gd, (1,),
                              mode=lax.GatherScatterMode.PROMISE_IN_BOUNDS)

        pltpu.sync_copy(idx_hbm.at[pl.ds(wid * CPW, CPW)], sd)
        r0 = sid * RPT
        pltpu.sync_copy(z_hbm.at[pl.ds(r0, RPT)], acc.at[pl.ds(r0, RPT)])
        plsc.subcore_barrier()

        def gathers(j, b):
            return (pltpu.make_async_copy(s_hbm.at[sd.at[j, 0]], srow.at[b], semS[b]),
                    pltpu.make_async_copy(d_hbm.at[sd.at[j, 1]], drow.at[b], semD[b]))

        def prime(j, b):
            g1, g2 = gathers(j, b)
            g1.start()
            g2.start()

        def edge_one(b, e):
            d = drow[b, e, pl.ds(0, 16)]
            vecs = []
            for g in range(NG):
                ab = srow[b, e, pl.ds(32 * g, 32)]
                va, vb = plsc.unpack(ab, format=plsc.PackFormat.INTERLEAVED)
                vecs += [va, vb]
            u = vecs[0] + d
            lu = jnp.where(u >= 0, u, 0.2 * u)
            bs = dg(d, bsidx)
            pm = jnp.where(headm, jnp.exp(lu - bs), 0.0)
            msg[b, e, pl.ds(DENC, 16)] = pm
            for k in range(NXV):
                ph = dg(pm, bcidx[k])
                msg[b, e, pl.ds(16 * k, 16)] = vecs[1 + k] * ph

        def half(j, b, wait_scat, do_prime):
            g1, g2 = gathers(j, b)
            g1.wait()
            g2.wait()
            if wait_scat:
                pltpu.make_async_copy(msg.at[b], acc.at[sd.at[j - 2, 1]],
                                      semW[b]).wait()
            lax.fori_loop(0, K // 4,
                          lambda i, c: [edge_one(b, i * 4 + r) for r in range(4)] and c,
                          0)
            pltpu.async_copy(msg.at[b], acc.at[sd.at[j, 1]], semW[b], add=True)
            if do_prime:
                prime(j + 2, b)

        prime(0, 0)
        prime(1, 1)
        half(0, 0, False, True)
        half(1, 1, False, True)

        def body2(jj, carry):
            half(2 * jj, 0, True, True)
            half(2 * jj + 1, 1, True, True)
            return carry

        lax.fori_loop(1, CPW // 2 - 1, body2, 0)
        half(CPW - 2, 0, True, False)
        half(CPW - 1, 1, True, False)
        pltpu.make_async_copy(msg.at[0], acc.at[sd.at[CPW - 2, 1]], semW[0]).wait()
        pltpu.make_async_copy(msg.at[1], acc.at[sd.at[CPW - 1, 1]], semW[1]).wait()

        plsc.subcore_barrier()
        pltpu.sync_copy(acc.at[pl.ds(r0, RPT)], out_hbm.at[cid, pl.ds(r0, RPT)])

    return sc_edge


def kernel(features, edge_index, W1, a1s, a1d, b1, W2, a2s, a2d, b2, W3, a3s, a3d, b3):
    E = edge_index.shape[1]
    E_tot = E + N
    E_pad = ((E_tot + 2 * NW * K - 1) // (2 * NW * K)) * (2 * NW * K)

    loop = jnp.arange(N, dtype=jnp.int32)
    padv = jnp.full((E_pad - E_tot,), N, jnp.int32)
    src = jnp.concatenate([edge_index[0].astype(jnp.int32), loop, padv])
    dst = jnp.concatenate([edge_index[1].astype(jnp.int32), loop, padv])
    idxp = jnp.concatenate([src.reshape(-1, 1, K), dst.reshape(-1, 1, K)], axis=1)

    def perm_m(DS):
        ng = DS // 32
        o = jnp.arange(DS)
        g, i = o // 32, o % 32
        n = jnp.where(i < 16, 32 * g + 2 * i, 32 * g + 2 * (i - 16) + 1)
        return jnp.zeros((DS, DS), jnp.float32).at[o, n].set(1.0)

    x0 = jnp.pad(features, ((0, NP - N), (0, 0)))
    z80 = jnp.zeros((NP, 80), jnp.float32)
    z64 = jnp.zeros((NP, 64), jnp.float32)

    prep1 = pl.pallas_call(
        functools.partial(_tc_prep1, H=8, HC=64, DS=96),
        out_shape=(jax.ShapeDtypeStruct((NP, 96), jnp.bfloat16),
                   jax.ShapeDtypeStruct((NP, 16), jnp.float32)),
    )
    S1, D1 = prep1(x0, W1, _expand_a(a1s), _expand_a(a1d), perm_m(96))

    sc12 = _make_sc_edge(E_pad, 96, 80, 4, 8, 8)
    acc1 = sc12(S1, D1, idxp, z80)

    mid2 = pl.pallas_call(
        functools.partial(_tc_mid, HCp=64, DENC=64, H=8, HC=64, DS=96),
        out_shape=(jax.ShapeDtypeStruct((NP, 96), jnp.bfloat16),
                   jax.ShapeDtypeStruct((NP, 16), jnp.float32)),
    )
    S2, D2 = mid2(acc1, _head_expand(8, 8), b1.reshape(1, 64), W2,
                  _expand_a(a2s), _expand_a(a2d), perm_m(96))
    acc2 = sc12(S2, D2, idxp, z80)

    mid3 = pl.pallas_call(
        functools.partial(_tc_mid, HCp=64, DENC=64, H=1, HC=40, DS=64),
        out_shape=(jax.ShapeDtypeStruct((NP, 64), jnp.bfloat16),
                   jax.ShapeDtypeStruct((NP, 16), jnp.float32)),
    )
    S3, D3 = mid3(acc2, _head_expand(8, 8), b2.reshape(1, 64), W3,
                  _expand_a(a3s), _expand_a(a3d), perm_m(64))

    sc3 = _make_sc_edge(E_pad, 64, 64, 3, 40, 1)
    acc3 = sc3(S3, D3, idxp, z64)

    final = pl.pallas_call(
        _tc_final,
        out_shape=jax.ShapeDtypeStruct((NP, 40), jnp.float32),
    )
    out = final(acc3, b3.reshape(1, 40))
    return out[:N]


# speedup vs baseline: 1.0265x; 1.0265x over previous
import functools

import jax
import jax.numpy as jnp
from jax import lax
from jax.experimental import pallas as pl
from jax.experimental.pallas import tpu as pltpu
from jax.experimental.pallas import tpu_sc as plsc

N = 10000
NP = 10112
NC, NS = 2, 16
NW = NC * NS
K = 192
NEG = -1e30


def _expand_a(a):
    H, C = a.shape
    M = jnp.zeros((H, C, H), jnp.float32).at[jnp.arange(H), :, jnp.arange(H)].set(a)
    return M.reshape(H * C, H)


def _head_expand(H, C):
    R = jnp.zeros((H, H, C), jnp.float32).at[jnp.arange(H), jnp.arange(H), :].set(1.0)
    return R.reshape(H, H * C)


def _prep_tables(xw, asrc, adst, H, HC, DS):
    row = lax.broadcasted_iota(jnp.int32, (NP, H), 0)
    asrc = jnp.where(row < N, asrc, NEG)
    gmax = jnp.max(asrc, axis=0, keepdims=True)
    t = gmax + adst
    bsub = jnp.where(t >= 0, t, 0.2 * t)
    def cat(parts):
        parts = [p for p in parts if p.shape[1] > 0]
        return parts[0] if len(parts) == 1 else jnp.concatenate(parts, axis=1)

    S = cat([asrc, jnp.zeros((NP, 16 - H), jnp.float32), xw,
             jnp.zeros((NP, DS - 16 - HC), jnp.float32)])
    D = cat([adst, bsub, jnp.zeros((NP, 16 - 2 * H), jnp.float32)])
    return S, D


def _tc_prep1(x_ref, w_ref, as_ref, ad_ref, s_ref, d_ref, *, H, HC, DS):
    xw = jnp.dot(x_ref[...], w_ref[...], preferred_element_type=jnp.float32)
    asrc = jnp.dot(xw, as_ref[...], preferred_element_type=jnp.float32)
    adst = jnp.dot(xw, ad_ref[...], preferred_element_type=jnp.float32)
    S, D = _prep_tables(xw, asrc, adst, H, HC, DS)
    s_ref[...] = S
    d_ref[...] = D


def _tc_mid(acc_ref, r_ref, b_ref, w_ref, as_ref, ad_ref, s_ref, d_ref,
            *, HCp, DENC, H, HC, DS):
    msum = acc_ref[0, :, 0:HCp] + acc_ref[1, :, 0:HCp]
    den = acc_ref[0, :, DENC:DENC + 8] + acc_ref[1, :, DENC:DENC + 8]
    den_e = jnp.dot(den, r_ref[...], preferred_element_type=jnp.float32)
    x = msum / (den_e + 1e-16) + b_ref[...]
    xw = jnp.dot(x, w_ref[...], preferred_element_type=jnp.float32)
    asrc = jnp.dot(xw, as_ref[...], preferred_element_type=jnp.float32)
    adst = jnp.dot(xw, ad_ref[...], preferred_element_type=jnp.float32)
    S, D = _prep_tables(xw, asrc, adst, H, HC, DS)
    s_ref[...] = S
    d_ref[...] = D


def _tc_final(acc_ref, b_ref, o_ref):
    msum = acc_ref[0, :, 0:40] + acc_ref[1, :, 0:40]
    den = acc_ref[0, :, 48:49] + acc_ref[1, :, 48:49]
    x = msum / (den + 1e-16) + b_ref[...]
    m = jnp.max(x, axis=1, keepdims=True)
    o_ref[...] = x - (m + jnp.log(jnp.sum(jnp.exp(x - m), axis=1, keepdims=True)))


def _make_sc_edge(E_pad, DS, DA, NXV, CW, HL):
    NG = DS // 32
    CPW = E_pad // (NW * K)
    RPT = NP // NS
    DENC = NXV * 16
    mesh = plsc.VectorSubcoreMesh(core_axis_name="c", subcore_axis_name="s")

    @functools.partial(
        pl.kernel,
        out_type=jax.ShapeDtypeStruct((NC, NP, DA), jnp.float32),
        mesh=mesh,
        compiler_params=pltpu.CompilerParams(use_tc_tiling_on_sc=False,
                                             needs_layout_passes=False),
        scratch_types=[
            pltpu.VMEM((CPW, 2, K), jnp.int32),
            pltpu.VMEM((2, K, DS), jnp.bfloat16),
            pltpu.VMEM((2, K, 16), jnp.float32),
            pltpu.VMEM((2, K, DA), jnp.float32),
            pltpu.VMEM_SHARED((NP, DA), jnp.float32),
            pltpu.SemaphoreType.DMA,
            pltpu.SemaphoreType.DMA,
            pltpu.SemaphoreType.DMA,
            pltpu.SemaphoreType.DMA,
            pltpu.SemaphoreType.DMA,
            pltpu.SemaphoreType.DMA,
        ],
    )
    def sc_edge(s_hbm, d_hbm, idx_hbm, z_hbm, out_hbm,
                sd, srow, drow, msg, acc, ss0, ss1, sd0, sd1, sw0, sw1):
        cid = lax.axis_index("c")
        sid = lax.axis_index("s")
        wid = cid * NS + sid
        lane = lax.iota(jnp.int32, 16)
        bsidx = HL + lane % HL
        bcidx = [(16 * k + lane) // CW for k in range(NXV)]
        headm = lane < HL
        semS = [ss0, ss1]
        semD = [sd0, sd1]
        semW = [sw0, sw1]
        gd = lax.GatherDimensionNumbers(offset_dims=(), collapsed_slice_dims=(0,),
                                        start_index_map=(0,))

        def dg(x, idx):
            return lax.gather(x, idx[:, None], gd, (1,),
                              mode=lax.GatherScatterMode.PROMISE_IN_BOUNDS)

        pltpu.sync_copy(idx_hbm.at[pl.ds(wid * CPW, CPW)], sd)
        r0 = sid * RPT
        pltpu.sync_copy(z_hbm.at[pl.ds(r0, RPT)], acc.at[pl.ds(r0, RPT)])
        plsc.subcore_barrier()

        def gathers(j, b):
            return (pltpu.make_async_copy(s_hbm.at[sd.at[j, 0]], srow.at[b], semS[b]),
                    pltpu.make_async_copy(d_hbm.at[sd.at[j, 1]], drow.at[b], semD[b]))

        def prime(j, b):
            g1, g2 = gathers(j, b)
            g1.start()
            g2.start()

        def edge_one(b, e):
            d = drow[b, e, pl.ds(0, 16)]
            vecs = []
            for g in range(NG):
                ab = srow[b, e, pl.ds(32 * g, 32)]
                va, vb = plsc.unpack(ab, format=plsc.PackFormat.INTERLEAVED)
                vecs += [va, vb]
            u = vecs[0] + d
            lu = jnp.where(u >= 0, u, 0.2 * u)
            bs = dg(d, bsidx)
            pm = jnp.where(headm, jnp.exp(lu - bs), 0.0)
            msg[b, e, pl.ds(DENC, 16)] = pm
            for k in range(NXV):
                ph = dg(pm, bcidx[k])
                msg[b, e, pl.ds(16 * k, 16)] = vecs[1 + k] * ph

        def half(j, b, wait_scat, do_prime):
            g1, g2 = gathers(j, b)
            g1.wait()
            g2.wait()
            if wait_scat:
                pltpu.make_async_copy(msg.at[b], acc.at[sd.at[j - 2, 1]],
                                      semW[b]).wait()
            lax.fori_loop(0, K // 4,
                          lambda i, c: [edge_one(b, i * 4 + r) for r in range(4)] and c,
                          0)
            pltpu.async_copy(msg.at[b], acc.at[sd.at[j, 1]], semW[b], add=True)
            if do_prime:
                prime(j + 2, b)

        prime(0, 0)
        prime(1, 1)
        half(0, 0, False, True)
        half(1, 1, False, True)

        def body2(jj, carry):
            half(2 * jj, 0, True, True)
            half(2 * jj + 1, 1, True, True)
            return carry

        lax.fori_loop(1, CPW // 2 - 1, body2, 0)
        half(CPW - 2, 0, True, False)
        half(CPW - 1, 1, True, False)
        pltpu.make_async_copy(msg.at[0], acc.at[sd.at[CPW - 2, 1]], semW[0]).wait()
        pltpu.make_async_copy(msg.at[1], acc.at[sd.at[CPW - 1, 1]], semW[1]).wait()

        plsc.subcore_barrier()
        pltpu.sync_copy(acc.at[pl.ds(r0, RPT)], out_hbm.at[cid, pl.ds(r0, RPT)])

    return sc_edge


def kernel(features, edge_index, W1, a1s, a1d, b1, W2, a2s, a2d, b2, W3, a3s, a3d, b3):
    E = edge_index.shape[1]
    E_tot = E + N
    E_pad = ((E_tot + 2 * NW * K - 1) // (2 * NW * K)) * (2 * NW * K)

    loop = jnp.arange(N, dtype=jnp.int32)
    padv = jnp.full((E_pad - E_tot,), N, jnp.int32)
    src = jnp.concatenate([edge_index[0].astype(jnp.int32), loop, padv])
    dst = jnp.concatenate([edge_index[1].astype(jnp.int32), loop, padv])
    idxp = jnp.concatenate([src.reshape(-1, 1, K), dst.reshape(-1, 1, K)], axis=1)

    def to_bf(S):
        ng = S.shape[1] // 32
        return (S.reshape(NP, ng, 2, 16).transpose(0, 1, 3, 2)
                .reshape(NP, ng * 32).astype(jnp.bfloat16))

    x0 = jnp.pad(features, ((0, NP - N), (0, 0)))
    z80 = jnp.zeros((NP, 80), jnp.float32)
    z64 = jnp.zeros((NP, 64), jnp.float32)

    prep1 = pl.pallas_call(
        functools.partial(_tc_prep1, H=8, HC=64, DS=96),
        out_shape=(jax.ShapeDtypeStruct((NP, 96), jnp.float32),
                   jax.ShapeDtypeStruct((NP, 16), jnp.float32)),
    )
    S1, D1 = prep1(x0, W1, _expand_a(a1s), _expand_a(a1d))

    sc12 = _make_sc_edge(E_pad, 96, 80, 4, 8, 8)
    acc1 = sc12(to_bf(S1), D1, idxp, z80)

    mid2 = pl.pallas_call(
        functools.partial(_tc_mid, HCp=64, DENC=64, H=8, HC=64, DS=96),
        out_shape=(jax.ShapeDtypeStruct((NP, 96), jnp.float32),
                   jax.ShapeDtypeStruct((NP, 16), jnp.float32)),
    )
    S2, D2 = mid2(acc1, _head_expand(8, 8), b1.reshape(1, 64), W2,
                  _expand_a(a2s), _expand_a(a2d))
    acc2 = sc12(to_bf(S2), D2, idxp, z80)

    mid3 = pl.pallas_call(
        functools.partial(_tc_mid, HCp=64, DENC=64, H=1, HC=40, DS=64),
        out_shape=(jax.ShapeDtypeStruct((NP, 64), jnp.float32),
                   jax.ShapeDtypeStruct((NP, 16), jnp.float32)),
    )
    S3, D3 = mid3(acc2, _head_expand(8, 8), b2.reshape(1, 64), W3,
                  _expand_a(a3s), _expand_a(a3d))

    sc3 = _make_sc_edge(E_pad, 64, 64, 3, 40, 1)
    acc3 = sc3(to_bf(S3), D3, idxp, z64)

    final = pl.pallas_call(
        _tc_final,
        out_shape=jax.ShapeDtypeStruct((NP, 40), jnp.float32),
    )
    out = final(acc3, b3.reshape(1, 40))
    return out[:N]


# speedup vs baseline: 1.0341x; 1.0074x over previous
import functools

import jax
import jax.numpy as jnp
from jax import lax
from jax.experimental import pallas as pl
from jax.experimental.pallas import tpu as pltpu
from jax.experimental.pallas import tpu_sc as plsc

N = 10000
NP = 10112
NC, NS = 2, 16
NW = NC * NS
K = 192
NEG = -1e30


def _expand_a(a):
    H, C = a.shape
    M = jnp.zeros((H, C, H), jnp.float32).at[jnp.arange(H), :, jnp.arange(H)].set(a)
    return M.reshape(H * C, H)


def _head_expand(H, C):
    R = jnp.zeros((H, H, C), jnp.float32).at[jnp.arange(H), jnp.arange(H), :].set(1.0)
    return R.reshape(H, H * C)


def _prep_tables(xw, asrc, adst, H, HC, DS):
    row = lax.broadcasted_iota(jnp.int32, (NP, H), 0)
    asrc = jnp.where(row < N, asrc, NEG)
    gmax = jnp.max(asrc, axis=0, keepdims=True)
    t = gmax + adst
    bsub = jnp.where(t >= 0, t, 0.2 * t)
    def cat(parts):
        parts = [p for p in parts if p.shape[1] > 0]
        return parts[0] if len(parts) == 1 else jnp.concatenate(parts, axis=1)

    S = cat([asrc, jnp.zeros((NP, 16 - H), jnp.float32), xw,
             jnp.zeros((NP, DS - 16 - HC), jnp.float32)])
    D = cat([adst, bsub, jnp.zeros((NP, 16 - 2 * H), jnp.float32)])
    return S, D


def _tc_prep1(x_ref, w_ref, as_ref, ad_ref, s_ref, d_ref, *, H, HC, DS):
    xw = jnp.dot(x_ref[...], w_ref[...], preferred_element_type=jnp.float32)
    asrc = jnp.dot(xw, as_ref[...], preferred_element_type=jnp.float32)
    adst = jnp.dot(xw, ad_ref[...], preferred_element_type=jnp.float32)
    S, D = _prep_tables(xw, asrc, adst, H, HC, DS)
    s_ref[...] = S
    d_ref[...] = D


def _tc_mid(acc_ref, r_ref, b_ref, w_ref, as_ref, ad_ref, s_ref, d_ref,
            *, HCp, DENC, H, HC, DS):
    msum = acc_ref[0, :, 0:HCp] + acc_ref[1, :, 0:HCp]
    den = acc_ref[0, :, DENC:DENC + 8] + acc_ref[1, :, DENC:DENC + 8]
    den_e = jnp.dot(den, r_ref[...], preferred_element_type=jnp.float32)
    x = msum / (den_e + 1e-16) + b_ref[...]
    xw = jnp.dot(x, w_ref[...], preferred_element_type=jnp.float32)
    asrc = jnp.dot(xw, as_ref[...], preferred_element_type=jnp.float32)
    adst = jnp.dot(xw, ad_ref[...], preferred_element_type=jnp.float32)
    S, D = _prep_tables(xw, asrc, adst, H, HC, DS)
    s_ref[...] = S
    d_ref[...] = D


def _tc_final(acc_ref, b_ref, o_ref):
    msum = acc_ref[0, :, 0:40] + acc_ref[1, :, 0:40]
    den = acc_ref[0, :, 48:49] + acc_ref[1, :, 48:49]
    x = msum / (den + 1e-16) + b_ref[...]
    m = jnp.max(x, axis=1, keepdims=True)
    o_ref[...] = x - (m + jnp.log(jnp.sum(jnp.exp(x - m), axis=1, keepdims=True)))


def _make_sc_edge(E_pad, DS, DA, NXV, CW, HL):
    NG = DS // 32
    CPW = E_pad // (NW * K)
    RPT = NP // NS
    DENC = NXV * 16
    mesh = plsc.VectorSubcoreMesh(core_axis_name="c", subcore_axis_name="s")

    @functools.partial(
        pl.kernel,
        out_type=jax.ShapeDtypeStruct((NC, NP, DA), jnp.float32),
        mesh=mesh,
        compiler_params=pltpu.CompilerParams(use_tc_tiling_on_sc=False,
                                             needs_layout_passes=False),
        scratch_types=[
            pltpu.VMEM((CPW, 2, K), jnp.int32),
            pltpu.VMEM((2, K, DS), jnp.bfloat16),
            pltpu.VMEM((2, K, 16), jnp.float32),
            pltpu.VMEM((2, K, DA), jnp.float32),
            pltpu.VMEM_SHARED((NP, DA), jnp.float32),
            pltpu.SemaphoreType.DMA,
            pltpu.SemaphoreType.DMA,
            pltpu.SemaphoreType.DMA,
            pltpu.SemaphoreType.DMA,
            pltpu.SemaphoreType.DMA,
            pltpu.SemaphoreType.DMA,
        ],
    )
    def sc_edge(s_hbm, d_hbm, idx_hbm, out_hbm,
                sd, srow, drow, msg, acc, ss0, ss1, sd0, sd1, sw0, sw1):
        cid = lax.axis_index("c")
        sid = lax.axis_index("s")
        wid = cid * NS + sid
        lane = lax.iota(jnp.int32, 16)
        bsidx = HL + lane % HL
        bcidx = [(16 * k + lane) // CW for k in range(NXV)]
        headm = lane < HL
        semS = [ss0, ss1]
        semD = [sd0, sd1]
        semW = [sw0, sw1]
        gd = lax.GatherDimensionNumbers(offset_dims=(), collapsed_slice_dims=(0,),
                                        start_index_map=(0,))

        def dg(x, idx):
            return lax.gather(x, idx[:, None], gd, (1,),
                              mode=lax.GatherScatterMode.PROMISE_IN_BOUNDS)

        pltpu.sync_copy(idx_hbm.at[pl.ds(wid * CPW, CPW)], sd)
        zv = jnp.zeros((16,), jnp.float32)

        def zero_body(e, c):
            for b in range(2):
                for k in range(DA // 16):
                    msg[b, e, pl.ds(16 * k, 16)] = zv
            return c

        lax.fori_loop(0, K, zero_body, 0)
        r0 = sid * RPT
        nfull = RPT // K
        for m in range(nfull):
            pltpu.sync_copy(msg.at[m % 2], acc.at[pl.ds(r0 + K * m, K)])
        rem = RPT - nfull * K
        if rem:
            pltpu.sync_copy(msg.at[0, pl.ds(0, rem)],
                            acc.at[pl.ds(r0 + K * nfull, rem)])
        plsc.subcore_barrier()

        def gathers(j, b):
            return (pltpu.make_async_copy(s_hbm.at[sd.at[j, 0]], srow.at[b], semS[b]),
                    pltpu.make_async_copy(d_hbm.at[sd.at[j, 1]], drow.at[b], semD[b]))

        def prime(j, b):
            g1, g2 = gathers(j, b)
            g1.start()
            g2.start()

        def edge_one(b, e):
            d = drow[b, e, pl.ds(0, 16)]
            vecs = []
            for g in range(NG):
                ab = srow[b, e, pl.ds(32 * g, 32)]
                va, vb = plsc.unpack(ab, format=plsc.PackFormat.INTERLEAVED)
                vecs += [va, vb]
            u = vecs[0] + d
            lu = jnp.where(u >= 0, u, 0.2 * u)
            bs = dg(d, bsidx)
            pm = jnp.where(headm, jnp.exp(lu - bs), 0.0)
            msg[b, e, pl.ds(DENC, 16)] = pm
            for k in range(NXV):
                ph = dg(pm, bcidx[k])
                msg[b, e, pl.ds(16 * k, 16)] = vecs[1 + k] * ph

        def half(j, b, wait_scat, do_prime):
            g1, g2 = gathers(j, b)
            g1.wait()
            g2.wait()
            if wait_scat:
                pltpu.make_async_copy(msg.at[b], acc.at[sd.at[j - 2, 1]],
                                      semW[b]).wait()
            lax.fori_loop(0, K // 4,
                          lambda i, c: [edge_one(b, i * 4 + r) for r in range(4)] and c,
                          0)
            pltpu.async_copy(msg.at[b], acc.at[sd.at[j, 1]], semW[b], add=True)
            if do_prime:
                prime(j + 2, b)

        prime(0, 0)
        prime(1, 1)
        half(0, 0, False, True)
        half(1, 1, False, True)

        def body2(jj, carry):
            half(2 * jj, 0, True, True)
            half(2 * jj + 1, 1, True, True)
            return carry

        lax.fori_loop(1, CPW // 2 - 1, body2, 0)
        half(CPW - 2, 0, True, False)
        half(CPW - 1, 1, True, False)
        pltpu.make_async_copy(msg.at[0], acc.at[sd.at[CPW - 2, 1]], semW[0]).wait()
        pltpu.make_async_copy(msg.at[1], acc.at[sd.at[CPW - 1, 1]], semW[1]).wait()

        plsc.subcore_barrier()
        pltpu.sync_copy(acc.at[pl.ds(r0, RPT)], out_hbm.at[cid, pl.ds(r0, RPT)])

    return sc_edge


def kernel(features, edge_index, W1, a1s, a1d, b1, W2, a2s, a2d, b2, W3, a3s, a3d, b3):
    E = edge_index.shape[1]
    E_tot = E + N
    E_pad = ((E_tot + 2 * NW * K - 1) // (2 * NW * K)) * (2 * NW * K)

    loop = jnp.arange(N, dtype=jnp.int32)
    padv = jnp.full((E_pad - E_tot,), N, jnp.int32)
    src = jnp.concatenate([edge_index[0].astype(jnp.int32), loop, padv])
    dst = jnp.concatenate([edge_index[1].astype(jnp.int32), loop, padv])
    idxp = jnp.concatenate([src.reshape(-1, 1, K), dst.reshape(-1, 1, K)], axis=1)

    def to_bf(S):
        ng = S.shape[1] // 32
        return (S.reshape(NP, ng, 2, 16).transpose(0, 1, 3, 2)
                .reshape(NP, ng * 32).astype(jnp.bfloat16))

    x0 = jnp.pad(features, ((0, NP - N), (0, 0)))

    prep1 = pl.pallas_call(
        functools.partial(_tc_prep1, H=8, HC=64, DS=96),
        out_shape=(jax.ShapeDtypeStruct((NP, 96), jnp.float32),
                   jax.ShapeDtypeStruct((NP, 16), jnp.float32)),
    )
    S1, D1 = prep1(x0, W1, _expand_a(a1s), _expand_a(a1d))

    sc12 = _make_sc_edge(E_pad, 96, 80, 4, 8, 8)
    acc1 = sc12(to_bf(S1), D1, idxp)

    mid2 = pl.pallas_call(
        functools.partial(_tc_mid, HCp=64, DENC=64, H=8, HC=64, DS=96),
        out_shape=(jax.ShapeDtypeStruct((NP, 96), jnp.float32),
                   jax.ShapeDtypeStruct((NP, 16), jnp.float32)),
    )
    S2, D2 = mid2(acc1, _head_expand(8, 8), b1.reshape(1, 64), W2,
                  _expand_a(a2s), _expand_a(a2d))
    acc2 = sc12(to_bf(S2), D2, idxp)

    mid3 = pl.pallas_call(
        functools.partial(_tc_mid, HCp=64, DENC=64, H=1, HC=40, DS=64),
        out_shape=(jax.ShapeDtypeStruct((NP, 64), jnp.float32),
                   jax.ShapeDtypeStruct((NP, 16), jnp.float32)),
    )
    S3, D3 = mid3(acc2, _head_expand(8, 8), b2.reshape(1, 64), W3,
                  _expand_a(a3s), _expand_a(a3d))

    sc3 = _make_sc_edge(E_pad, 64, 64, 3, 40, 1)
    acc3 = sc3(to_bf(S3), D3, idxp)

    final = pl.pallas_call(
        _tc_final,
        out_shape=jax.ShapeDtypeStruct((NP, 40), jnp.float32),
    )
    out = final(acc3, b3.reshape(1, 40))
    return out[:N]
